# baseline (speedup 1.0000x reference)
import functools

import jax
import jax.numpy as jnp
from jax import lax
from jax.experimental import pallas as pl
from jax.experimental.pallas import tpu as pltpu
from jax.experimental.pallas import tpu_sc as plsc

EMBED_DIM = 32
NUM_PARTICLES = 4
LANES = 16


def _sort4(a0, a1, a2, a3):
    b0 = jnp.minimum(a0, a1)
    b1 = jnp.maximum(a0, a1)
    b2 = jnp.minimum(a2, a3)
    b3 = jnp.maximum(a2, a3)
    c0 = jnp.minimum(b0, b2)
    c2 = jnp.maximum(b0, b2)
    c1 = jnp.minimum(b1, b3)
    c3 = jnp.maximum(b1, b3)
    d1 = jnp.minimum(c1, c2)
    d2 = jnp.maximum(c1, c2)
    return c0, d1, d2, c3


def _sqrt_nr(s):
    i = lax.bitcast_convert_type(s, jnp.int32)
    y = lax.bitcast_convert_type(
        jnp.int32(0x5F3759DF) - lax.shift_right_arithmetic(i, 1), jnp.float32
    )
    for _ in range(3):
        t = s * y
        y = y * (1.5 - 0.5 * t * y)
    return s * y


def _make_sc_kernel(batch, ent_dim, num_rel, rel_dim):
    info = plsc.get_sparse_core_info()
    nc, ns = info.num_cores, info.num_subcores
    nw = nc * ns
    assert batch % (8 * nw) == 0
    bpw = batch // nw
    n_chunks = bpw // LANES
    mesh = plsc.VectorSubcoreMesh(core_axis_name="c", subcore_axis_name="s")

    @functools.partial(
        pl.kernel,
        mesh=mesh,
        out_type=jax.ShapeDtypeStruct((batch,), jnp.float32),
        compiler_params=pltpu.CompilerParams(needs_layout_passes=False),
        scratch_types=[
            pltpu.VMEM((2, bpw // 2), jnp.int32),
            pltpu.VMEM((2, bpw // 2), jnp.int32),
            pltpu.VMEM((bpw,), jnp.int32),
            pltpu.VMEM((2, bpw // 2), jnp.int32),
            pltpu.VMEM((bpw, ent_dim), jnp.float32),
            pltpu.VMEM((bpw, ent_dim), jnp.float32),
            pltpu.VMEM((bpw, 128), jnp.float32),
            pltpu.VMEM((bpw,), jnp.float32),
            pltpu.SemaphoreType.DMA,
            pltpu.SemaphoreType.DMA,
        ],
    )
    def sc_kernel(head_hbm, rel_hbm, tail_hbm, ent_hbm, relt_hbm, out_hbm,
                  hidx_v, tidx_v, ridx_v, rq_v, hrows_v, trows_v, rrows_v,
                  out_v, sem_a, sem_b):
        wid = lax.axis_index("s") * nc + lax.axis_index("c")
        base = wid * bpw
        half = bpw // 2
        pack_shift = (128 // rel_dim).bit_length() - 1

        with jax.named_scope("dma_phase"):
            pltpu.sync_copy(head_hbm.at[pl.ds(base, half)], hidx_v.at[0])
            pltpu.sync_copy(head_hbm.at[pl.ds(base + half, half)],
                            hidx_v.at[1])
            pltpu.sync_copy(tail_hbm.at[pl.ds(base, half)], tidx_v.at[0])
            pltpu.sync_copy(tail_hbm.at[pl.ds(base + half, half)],
                            tidx_v.at[1])
            pltpu.sync_copy(rel_hbm.at[pl.ds(base, bpw)], ridx_v)
            for s in range(2):
                for c in range(half // LANES):
                    o = s * half + c * LANES
                    rq_v[s, pl.ds(c * LANES, LANES)] = (
                        lax.shift_right_arithmetic(
                            ridx_v[pl.ds(o, LANES)], pack_shift))
            cps = []
            for s, sem in ((0, sem_a), (1, sem_b)):
                dst = pl.ds(s * half, half)
                cps.append([
                    pltpu.async_copy(ent_hbm.at[hidx_v.at[s]],
                                     hrows_v.at[dst], sem),
                    pltpu.async_copy(ent_hbm.at[tidx_v.at[s]],
                                     trows_v.at[dst], sem),
                    pltpu.async_copy(relt_hbm.at[rq_v.at[s]],
                                     rrows_v.at[dst], sem),
                ])
            for cp in cps[0]:
                cp.wait()

        lane = lax.broadcasted_iota(jnp.int32, (LANES,), 0)

        pack = 128 // rel_dim

        lsh2 = lax.shift_right_arithmetic(lane, 2)
        kg = [(g + lane) & jnp.int32(NUM_PARTICLES - 1)
              for g in range(NUM_PARTICLES)]

        def chunk_body(c, carry):
            row = c * LANES + lane
            rv = ridx_v[pl.ds(c * LANES, LANES)]
            cb = (rv & jnp.int32(pack - 1)) * rel_dim
            acc = jnp.zeros((LANES,), jnp.float32)
            for j in range(EMBED_DIM):
                dvec = (j + lsh2) & jnp.int32(EMBED_DIM - 1)
                colbase = lax.shift_left(dvec, 2)
                rl = plsc.load_gather(rrows_v, [row, cb + dvec])
                cols = [colbase + kg[g] for g in range(NUM_PARTICLES)]
                h = [plsc.load_gather(hrows_v, [row, cols[g]])
                     for g in range(NUM_PARTICLES)]
                t = [plsc.load_gather(trows_v, [row, cols[g]])
                     for g in range(NUM_PARTICLES)]
                hs = _sort4(*h)
                ts = _sort4(*t)
                ssq = jnp.zeros((LANES,), jnp.float32)
                for k in range(NUM_PARTICLES):
                    dk = hs[k] - ts[k] + rl
                    ssq = ssq + dk * dk
                acc = acc + _sqrt_nr(ssq)
            out_v[pl.ds(c * LANES, LANES)] = -acc
            return carry

        with jax.named_scope("compute_phase"):
            lax.fori_loop(0, n_chunks // 2, chunk_body, 0)
            for cp in cps[1]:
                cp.wait()
            lax.fori_loop(n_chunks // 2, n_chunks, chunk_body, 0)
        with jax.named_scope("writeback"):
            pltpu.sync_copy(out_v, out_hbm.at[pl.ds(base, bpw)])

    return sc_kernel


def kernel(head, rel, tail, entity_table, relation_table):
    batch = head.shape[0]
    ent_dim = entity_table.shape[1]
    num_rel, rel_dim = relation_table.shape
    relt_packed = relation_table.reshape(num_rel * rel_dim // 128, 128)
    sc = _make_sc_kernel(batch, ent_dim, num_rel, rel_dim)
    return sc(head, rel, tail, entity_table, relt_packed)


# speedup vs baseline: 1.0768x; 1.0768x over previous
import functools

import jax
import jax.numpy as jnp
from jax import lax
from jax.experimental import pallas as pl
from jax.experimental.pallas import tpu as pltpu
from jax.experimental.pallas import tpu_sc as plsc

EMBED_DIM = 32
NUM_PARTICLES = 4
LANES = 16


def _sort4(a0, a1, a2, a3):
    b0 = jnp.minimum(a0, a1)
    b1 = jnp.maximum(a0, a1)
    b2 = jnp.minimum(a2, a3)
    b3 = jnp.maximum(a2, a3)
    c0 = jnp.minimum(b0, b2)
    c2 = jnp.maximum(b0, b2)
    c1 = jnp.minimum(b1, b3)
    c3 = jnp.maximum(b1, b3)
    d1 = jnp.minimum(c1, c2)
    d2 = jnp.maximum(c1, c2)
    return c0, d1, d2, c3


def _sqrt_nr(s):
    i = lax.bitcast_convert_type(s, jnp.int32)
    y = lax.bitcast_convert_type(
        jnp.int32(0x5F3759DF) - lax.shift_right_arithmetic(i, 1), jnp.float32
    )
    for _ in range(2):
        t = s * y
        y = y * (1.5 - 0.5 * t * y)
    return s * y


def _make_sc_kernel(batch, ent_dim, num_rel, rel_dim):
    info = plsc.get_sparse_core_info()
    nc, ns = info.num_cores, info.num_subcores
    nw = nc * ns
    assert batch % (8 * nw) == 0
    bpw = batch // nw
    n_chunks = bpw // LANES
    mesh = plsc.VectorSubcoreMesh(core_axis_name="c", subcore_axis_name="s")

    @functools.partial(
        pl.kernel,
        mesh=mesh,
        out_type=jax.ShapeDtypeStruct((batch,), jnp.float32),
        compiler_params=pltpu.CompilerParams(needs_layout_passes=False),
        scratch_types=[
            pltpu.VMEM((2, bpw // 2), jnp.int32),
            pltpu.VMEM((2, bpw // 2), jnp.int32),
            pltpu.VMEM((bpw,), jnp.int32),
            pltpu.VMEM((2, bpw // 2), jnp.int32),
            pltpu.VMEM((bpw, ent_dim), jnp.float32),
            pltpu.VMEM((bpw, ent_dim), jnp.float32),
            pltpu.VMEM((bpw, 128), jnp.float32),
            pltpu.VMEM((bpw,), jnp.float32),
            pltpu.SemaphoreType.DMA,
            pltpu.SemaphoreType.DMA,
            pltpu.SemaphoreType.DMA,
        ],
    )
    def sc_kernel(head_hbm, rel_hbm, tail_hbm, ent_hbm, relt_hbm, out_hbm,
                  hidx_v, tidx_v, ridx_v, rq_v, hrows_v, trows_v, rrows_v,
                  out_v, sem_a, sem_b, sem_i):
        wid = lax.axis_index("s") * nc + lax.axis_index("c")
        base = wid * bpw
        half = bpw // 2
        pack_shift = (128 // rel_dim).bit_length() - 1

        with jax.named_scope("dma_phase"):
            icps = [
                pltpu.async_copy(head_hbm.at[pl.ds(base, half)],
                                 hidx_v.at[0], sem_i),
                pltpu.async_copy(head_hbm.at[pl.ds(base + half, half)],
                                 hidx_v.at[1], sem_i),
                pltpu.async_copy(tail_hbm.at[pl.ds(base, half)],
                                 tidx_v.at[0], sem_i),
                pltpu.async_copy(tail_hbm.at[pl.ds(base + half, half)],
                                 tidx_v.at[1], sem_i),
                pltpu.async_copy(rel_hbm.at[pl.ds(base, bpw)], ridx_v,
                                 sem_i),
            ]
            for cp in icps:
                cp.wait()
            cps = []
            for s, sem in ((0, sem_a), (1, sem_b)):
                dst = pl.ds(s * half, half)
                cps.append([
                    pltpu.async_copy(ent_hbm.at[hidx_v.at[s]],
                                     hrows_v.at[dst], sem),
                    pltpu.async_copy(ent_hbm.at[tidx_v.at[s]],
                                     trows_v.at[dst], sem),
                ])
            for s in range(2):
                for c in range(half // LANES):
                    o = s * half + c * LANES
                    rq_v[s, pl.ds(c * LANES, LANES)] = (
                        lax.shift_right_arithmetic(
                            ridx_v[pl.ds(o, LANES)], pack_shift))
            for s, sem in ((0, sem_a), (1, sem_b)):
                dst = pl.ds(s * half, half)
                cps[s].append(
                    pltpu.async_copy(relt_hbm.at[rq_v.at[s]],
                                     rrows_v.at[dst], sem))
            for cp in cps[0]:
                cp.wait()

        lane = lax.broadcasted_iota(jnp.int32, (LANES,), 0)

        pack = 128 // rel_dim

        lsh2 = lax.shift_right_arithmetic(lane, 2)
        kg = [(g + lane) & jnp.int32(NUM_PARTICLES - 1)
              for g in range(NUM_PARTICLES)]

        def chunk_body(c, carry):
            row = c * LANES + lane
            rv = ridx_v[pl.ds(c * LANES, LANES)]
            cb = (rv & jnp.int32(pack - 1)) * rel_dim
            acc = jnp.zeros((LANES,), jnp.float32)
            for j in range(EMBED_DIM):
                dvec = (j + lsh2) & jnp.int32(EMBED_DIM - 1)
                colbase = lax.shift_left(dvec, 2)
                rl = plsc.load_gather(rrows_v, [row, cb + dvec])
                cols = [colbase + kg[g] for g in range(NUM_PARTICLES)]
                h = [plsc.load_gather(hrows_v, [row, cols[g]])
                     for g in range(NUM_PARTICLES)]
                t = [plsc.load_gather(trows_v, [row, cols[g]])
                     for g in range(NUM_PARTICLES)]
                hs = _sort4(*h)
                ts = _sort4(*t)
                ssq = jnp.zeros((LANES,), jnp.float32)
                for k in range(NUM_PARTICLES):
                    dk = hs[k] - ts[k] + rl
                    ssq = ssq + dk * dk
                acc = acc + _sqrt_nr(ssq)
            out_v[pl.ds(c * LANES, LANES)] = -acc
            return carry

        with jax.named_scope("compute_phase"):
            lax.fori_loop(0, n_chunks // 2, chunk_body, 0)
            for cp in cps[1]:
                cp.wait()
            lax.fori_loop(n_chunks // 2, n_chunks, chunk_body, 0)
        with jax.named_scope("writeback"):
            pltpu.sync_copy(out_v, out_hbm.at[pl.ds(base, bpw)])

    return sc_kernel


def kernel(head, rel, tail, entity_table, relation_table):
    batch = head.shape[0]
    ent_dim = entity_table.shape[1]
    num_rel, rel_dim = relation_table.shape
    relt_packed = relation_table.reshape(num_rel * rel_dim // 128, 128)
    sc = _make_sc_kernel(batch, ent_dim, num_rel, rel_dim)
    return sc(head, rel, tail, entity_table, relt_packed)


# speedup vs baseline: 1.0920x; 1.0141x over previous
import functools

import jax
import jax.numpy as jnp
from jax import lax
from jax.experimental import pallas as pl
from jax.experimental.pallas import tpu as pltpu
from jax.experimental.pallas import tpu_sc as plsc

EMBED_DIM = 32
NUM_PARTICLES = 4
LANES = 16


def _sort4(a0, a1, a2, a3):
    b0 = jnp.minimum(a0, a1)
    b1 = jnp.maximum(a0, a1)
    b2 = jnp.minimum(a2, a3)
    b3 = jnp.maximum(a2, a3)
    c0 = jnp.minimum(b0, b2)
    c2 = jnp.maximum(b0, b2)
    c1 = jnp.minimum(b1, b3)
    c3 = jnp.maximum(b1, b3)
    d1 = jnp.minimum(c1, c2)
    d2 = jnp.maximum(c1, c2)
    return c0, d1, d2, c3


def _sqrt_nr(s):
    i = lax.bitcast_convert_type(s, jnp.int32)
    y = lax.bitcast_convert_type(
        jnp.int32(0x5F3759DF) - lax.shift_right_arithmetic(i, 1), jnp.float32
    )
    t = s * y
    y = y * (1.5 - 0.5 * t * y)
    return s * y


def _make_sc_kernel(batch, ent_dim, num_rel, rel_dim):
    info = plsc.get_sparse_core_info()
    nc, ns = info.num_cores, info.num_subcores
    nw = nc * ns
    assert batch % (8 * nw) == 0
    bpw = batch // nw
    n_chunks = bpw // LANES
    mesh = plsc.VectorSubcoreMesh(core_axis_name="c", subcore_axis_name="s")

    @functools.partial(
        pl.kernel,
        mesh=mesh,
        out_type=jax.ShapeDtypeStruct((batch,), jnp.float32),
        compiler_params=pltpu.CompilerParams(needs_layout_passes=False),
        scratch_types=[
            pltpu.VMEM((2, bpw // 2), jnp.int32),
            pltpu.VMEM((2, bpw // 2), jnp.int32),
            pltpu.VMEM((bpw,), jnp.int32),
            pltpu.VMEM((2, bpw // 2), jnp.int32),
            pltpu.VMEM((bpw, ent_dim), jnp.float32),
            pltpu.VMEM((bpw, ent_dim), jnp.float32),
            pltpu.VMEM((bpw, 128), jnp.float32),
            pltpu.VMEM((bpw,), jnp.float32),
            pltpu.SemaphoreType.DMA,
            pltpu.SemaphoreType.DMA,
            pltpu.SemaphoreType.DMA,
            pltpu.SemaphoreType.DMA,
        ],
    )
    def sc_kernel(head_hbm, rel_hbm, tail_hbm, ent_hbm, relt_hbm, out_hbm,
                  hidx_v, tidx_v, ridx_v, rq_v, hrows_v, trows_v, rrows_v,
                  out_v, sem_a, sem_b, sem_i, sem_i2):
        wid = lax.axis_index("s") * nc + lax.axis_index("c")
        base = wid * bpw
        half = bpw // 2
        pack_shift = (128 // rel_dim).bit_length() - 1

        with jax.named_scope("dma_phase"):
            icps_a = [
                pltpu.async_copy(head_hbm.at[pl.ds(base, half)],
                                 hidx_v.at[0], sem_i),
                pltpu.async_copy(tail_hbm.at[pl.ds(base, half)],
                                 tidx_v.at[0], sem_i),
            ]
            icps_b = [
                pltpu.async_copy(head_hbm.at[pl.ds(base + half, half)],
                                 hidx_v.at[1], sem_i2),
                pltpu.async_copy(tail_hbm.at[pl.ds(base + half, half)],
                                 tidx_v.at[1], sem_i2),
                pltpu.async_copy(rel_hbm.at[pl.ds(base, bpw)], ridx_v,
                                 sem_i2),
            ]
            for cp in icps_a:
                cp.wait()
            cps = [[
                pltpu.async_copy(ent_hbm.at[hidx_v.at[0]],
                                 hrows_v.at[pl.ds(0, half)], sem_a),
                pltpu.async_copy(ent_hbm.at[tidx_v.at[0]],
                                 trows_v.at[pl.ds(0, half)], sem_a),
            ]]
            for cp in icps_b:
                cp.wait()
            cps.append([
                pltpu.async_copy(ent_hbm.at[hidx_v.at[1]],
                                 hrows_v.at[pl.ds(half, half)], sem_b),
                pltpu.async_copy(ent_hbm.at[tidx_v.at[1]],
                                 trows_v.at[pl.ds(half, half)], sem_b),
            ])
            for s in range(2):
                for c in range(half // LANES):
                    o = s * half + c * LANES
                    rq_v[s, pl.ds(c * LANES, LANES)] = (
                        lax.shift_right_arithmetic(
                            ridx_v[pl.ds(o, LANES)], pack_shift))
            for s, sem in ((0, sem_a), (1, sem_b)):
                dst = pl.ds(s * half, half)
                cps[s].append(
                    pltpu.async_copy(relt_hbm.at[rq_v.at[s]],
                                     rrows_v.at[dst], sem))
            for cp in cps[0]:
                cp.wait()

        lane = lax.broadcasted_iota(jnp.int32, (LANES,), 0)

        pack = 128 // rel_dim

        lsh2 = lax.shift_right_arithmetic(lane, 2)
        kg = [(g + lane) & jnp.int32(NUM_PARTICLES - 1)
              for g in range(NUM_PARTICLES)]

        def chunk_body(c, carry):
            row = c * LANES + lane
            rv = ridx_v[pl.ds(c * LANES, LANES)]
            cb = (rv & jnp.int32(pack - 1)) * rel_dim
            acc = jnp.zeros((LANES,), jnp.float32)
            for j in range(EMBED_DIM):
                dvec = (j + lsh2) & jnp.int32(EMBED_DIM - 1)
                colbase = lax.shift_left(dvec, 2)
                rl = plsc.load_gather(rrows_v, [row, cb + dvec])
                cols = [colbase + kg[g] for g in range(NUM_PARTICLES)]
                h = [plsc.load_gather(hrows_v, [row, cols[g]])
                     for g in range(NUM_PARTICLES)]
                t = [plsc.load_gather(trows_v, [row, cols[g]])
                     for g in range(NUM_PARTICLES)]
                hs = _sort4(*h)
                ts = _sort4(*t)
                ssq = jnp.zeros((LANES,), jnp.float32)
                for k in range(NUM_PARTICLES):
                    dk = hs[k] - ts[k] + rl
                    ssq = ssq + dk * dk
                acc = acc + _sqrt_nr(ssq)
            out_v[pl.ds(c * LANES, LANES)] = -acc
            return carry

        with jax.named_scope("compute_phase"):
            lax.fori_loop(0, n_chunks // 2, chunk_body, 0)
            for cp in cps[1]:
                cp.wait()
            lax.fori_loop(n_chunks // 2, n_chunks, chunk_body, 0)
        with jax.named_scope("writeback"):
            pltpu.sync_copy(out_v, out_hbm.at[pl.ds(base, bpw)])

    return sc_kernel


def kernel(head, rel, tail, entity_table, relation_table):
    batch = head.shape[0]
    ent_dim = entity_table.shape[1]
    num_rel, rel_dim = relation_table.shape
    relt_packed = relation_table.reshape(num_rel * rel_dim // 128, 128)
    sc = _make_sc_kernel(batch, ent_dim, num_rel, rel_dim)
    return sc(head, rel, tail, entity_table, relt_packed)
